# Initial kernel scaffold; baseline (speedup 1.0000x reference)
import jax
import jax.numpy as jnp
from jax.experimental import pallas as pl


def kernel(z, A, gamma, beta, W1, b1, W2, b2):
    raise NotImplementedError("write your pallas kernel here")


# speedup vs baseline: 1.2235x; 1.2235x over previous
import jax
import jax.numpy as jnp
from jax.experimental import pallas as pl
from jax.experimental.pallas import tpu as pltpu

NODE_DIM = 1024
NUM_TYPES = 16
BATCH = 256


def _mixer_body(at_ref, z_ref, g_ref, bt_ref, w1_ref, b1_ref, w2_ref, b2_ref,
                o_ref):
    k = pl.program_id(0)
    acc = at_ref[k, 0] * z_ref[:, 0, :]
    for j in range(1, NUM_TYPES):
        acc = acc + at_ref[k, j] * z_ref[:, j, :]
    mu = jnp.mean(acc, axis=1, keepdims=True)
    xc = acc - mu
    var = jnp.mean(xc * xc, axis=1, keepdims=True)
    azn = xc * jax.lax.rsqrt(var + 1e-5) * g_ref[0, :] + bt_ref[0, :]
    azb = azn.astype(jnp.bfloat16)
    h = jnp.dot(azb, w1_ref[0].astype(jnp.bfloat16),
                preferred_element_type=jnp.float32) + b1_ref[0, 0, :]
    h = jnp.where(h > 0, h, jnp.exp(h) - 1.0)
    mix = jnp.dot(h.astype(jnp.bfloat16), w2_ref[0].astype(jnp.bfloat16),
                  preferred_element_type=jnp.float32) + b2_ref[0, 0, :]
    o_ref[0, :, :] = mix + azn


def kernel(z, A, gamma, beta, W1, b1, W2, b2):
    K = NUM_TYPES
    d = NODE_DIM
    B = z.shape[0] // K
    zb = z.reshape(B, K, d)
    at = A.T
    g2 = gamma.reshape(1, d)
    bt2 = beta.reshape(1, d)
    b1r = b1.reshape(K, 1, 2 * d)
    b2r = b2.reshape(K, 1, d)

    out = pl.pallas_call(
        _mixer_body,
        grid=(K,),
        in_specs=[
            pl.BlockSpec(memory_space=pltpu.SMEM),
            pl.BlockSpec((B, K, d), lambda k: (0, 0, 0)),
            pl.BlockSpec((1, d), lambda k: (0, 0)),
            pl.BlockSpec((1, d), lambda k: (0, 0)),
            pl.BlockSpec((1, d, 2 * d), lambda k: (k, 0, 0)),
            pl.BlockSpec((1, 1, 2 * d), lambda k: (k, 0, 0)),
            pl.BlockSpec((1, 2 * d, d), lambda k: (k, 0, 0)),
            pl.BlockSpec((1, 1, d), lambda k: (k, 0, 0)),
        ],
        out_specs=pl.BlockSpec((1, B, d), lambda k: (k, 0, 0)),
        out_shape=jax.ShapeDtypeStruct((K, B, d), jnp.float32),
        compiler_params=pltpu.CompilerParams(
            dimension_semantics=("arbitrary",),
        ),
    )(at, zb, g2, bt2, W1, b1r, W2, b2r)
    return out.transpose(1, 0, 2).reshape(B * K, d)
